# baseline (speedup 1.0000x reference)
import functools

import jax
import jax.numpy as jnp
from jax import lax
from jax.experimental import pallas as pl
from jax.experimental.pallas import tpu as pltpu
from jax.experimental.pallas import tpu_sc as plsc

_NROW = 16384
_SEQ = 200
_D = 64
_BLK = 16
_SPLIT = 104

_NW = 32
_RPW = _NROW // _NW


def _rank_body(ts_ref, idx_ref):
    ts = ts_ref[...]
    b = ts.shape[0]
    tsi = lax.bitcast_convert_type(ts, jnp.int32)
    a_k = jnp.broadcast_to(tsi[:, :, None], (b, _SEQ, _SEQ))
    a_j = jnp.broadcast_to(tsi[:, None, :], (b, _SEQ, _SEQ))
    k_iota = lax.broadcasted_iota(jnp.int32, (b, _SEQ, _SEQ), 1)
    j_iota = lax.broadcasted_iota(jnp.int32, (b, _SEQ, _SEQ), 2)
    ltf = jnp.where(a_k < a_j, 1.0, 0.0)
    eqf = jnp.where(a_k == a_j, 1.0, 0.0)
    trif = jnp.where(k_iota < j_iota, 1.0, 0.0)
    cmp = ltf + eqf * trif
    rank = jnp.sum(cmp, axis=1).astype(jnp.int32)
    pid = pl.program_id(0)
    row = pid * b + lax.broadcasted_iota(jnp.int32, (b, _SEQ), 0)
    idx_ref[...] = rank + row * _SEQ


def _tc_rank(ts):
    return pl.pallas_call(
        _rank_body,
        grid=(_NROW // _BLK,),
        in_specs=[pl.BlockSpec((_BLK, _SEQ), lambda i: (i, 0))],
        out_specs=pl.BlockSpec((_BLK, _SEQ), lambda i: (i, 0)),
        out_shape=jax.ShapeDtypeStruct((_NROW, _SEQ), jnp.int32),
    )(ts)


def _sc_scatter_body(table_hbm, idx_hbm, out_hbm,
                     tab_a, tab_b, ia, ib, sem_a, sem_b):
    wid = lax.axis_index("s") * 2 + lax.axis_index("c")
    base = wid * _RPW * _SEQ
    pltpu.sync_copy(table_hbm.at[pl.ds(0, _SPLIT)], tab_a)
    pltpu.sync_copy(table_hbm.at[pl.ds(_SPLIT, _SEQ - _SPLIT)], tab_b)

    @pl.loop(0, _RPW)
    def _(i):
        off = base + i * _SEQ
        pltpu.sync_copy(idx_hbm.at[pl.ds(off, _SPLIT)], ia)
        pltpu.sync_copy(idx_hbm.at[pl.ds(off + _SPLIT, _SEQ - _SPLIT)], ib)
        ca = pltpu.async_copy(tab_a, out_hbm.at[ia], sem_a)
        cb = pltpu.async_copy(tab_b, out_hbm.at[ib], sem_b)
        ca.wait()
        cb.wait()


def _sc_scatter(pos_embeddings, idx_flat):
    mesh = plsc.VectorSubcoreMesh(core_axis_name="c", subcore_axis_name="s")
    k = pl.kernel(
        _sc_scatter_body,
        mesh=mesh,
        compiler_params=pltpu.CompilerParams(use_tc_tiling_on_sc=False),
        out_type=jax.ShapeDtypeStruct((_NROW * _SEQ, _D), jnp.float32),
        scratch_types=[
            pltpu.VMEM((_SPLIT, _D), jnp.float32),
            pltpu.VMEM((_SEQ - _SPLIT, _D), jnp.float32),
            pltpu.VMEM((_SPLIT,), jnp.int32),
            pltpu.VMEM((_SEQ - _SPLIT,), jnp.int32),
            pltpu.SemaphoreType.DMA,
            pltpu.SemaphoreType.DMA,
        ],
    )
    return k(pos_embeddings, idx_flat)


@jax.jit
def kernel(ts, pos_embeddings):
    idx = _tc_rank(ts)
    out_flat = _sc_scatter(pos_embeddings, idx.reshape(-1))
    return out_flat.reshape(_NROW, _SEQ, _D)
